# baseline (speedup 1.0000x reference)
import jax
import jax.numpy as jnp
from jax import lax
from jax.experimental import pallas as pl
from jax.experimental.pallas import tpu as pltpu, tpu_sc as plsc

N = 262144
F = 26
NBINS = 16
L = 16
PHASES = 13
TOTAL = N * F

COLS = 1024
ROWS = TOTAL // COLS
TC_BLOCK_ROWS = 416
SC_ROWS = 416
TC_ROWS = ROWS - SC_ROWS
TC_GRID = TC_ROWS // TC_BLOCK_ROWS

NWORK = 32
SC_TOTAL = SC_ROWS * COLS
SC_PER_W = SC_TOTAL // NWORK
PIECE = 3328
NP = SC_PER_W // PIECE


def _sc_kernel(x_hbm, edges_hbm, hi_hbm, out_hbm, xb, ob, ev, hv, insem, outsem):
    nc = lax.axis_size("c")
    wid = lax.axis_index("s") * nc + lax.axis_index("c")
    pltpu.sync_copy(edges_hbm, ev)
    pltpu.sync_copy(hi_hbm, hv)
    wbase = wid * SC_PER_W

    def compute_piece(buf, xbuf, obuf):
        def group_body(g, carry):
            goff = g * (PHASES * L)
            for p in range(PHASES):
                off = goff + p * L
                xv = xbuf[buf, pl.ds(off, L)]
                cnt = jnp.zeros((L,), jnp.int32)
                for b in range(1, NBINS):
                    cnt = cnt + jnp.where(xv >= ev[pl.ds((b * PHASES + p) * L, L)], 1, 0)
                idx = jnp.where(xv < hv[pl.ds(p * L, L)], cnt, 0)
                obuf[buf, pl.ds(off, L)] = idx
            return carry

        lax.fori_loop(0, PIECE // (PHASES * L), group_body, 0)

    in_h = [None] * NP
    out_h = [None] * NP
    in_h[0] = pltpu.async_copy(x_hbm.at[pl.ds(wbase, PIECE)], xb.at[0], insem)
    for r in range(NP):
        if r + 1 < NP:
            in_h[r + 1] = pltpu.async_copy(
                x_hbm.at[pl.ds(wbase + (r + 1) * PIECE, PIECE)],
                xb.at[(r + 1) % 2], insem)
        in_h[r].wait()
        if r >= 2:
            out_h[r - 2].wait()
        compute_piece(r % 2, xb, ob)
        out_h[r] = pltpu.async_copy(
            ob.at[r % 2], out_hbm.at[pl.ds(wbase + r * PIECE, PIECE)], outsem)
    for r in range(max(NP - 2, 0), NP):
        out_h[r].wait()


def _tc_kernel(x_ref, e_ref, hi_ref, o_ref):
    x = x_ref[...].reshape(TC_BLOCK_ROWS // PHASES, PHASES, COLS)
    cnt = jnp.zeros(x.shape, jnp.int32)
    for b in range(1, NBINS):
        cnt = cnt + jnp.where(x >= e_ref[b][None], 1, 0)
    idx = jnp.where(x < hi_ref[...][None], cnt, 0)
    o_ref[...] = idx.reshape(TC_BLOCK_ROWS, COLS).astype(jnp.int8)


def kernel(x, ge_tensor, lt_tensor):
    x2 = x.astype(jnp.float32).reshape(ROWS, COLS)

    ge_t = ge_tensor.T
    lt_last = lt_tensor[:, NBINS - 1]

    edges_sc = jnp.broadcast_to(
        ge_t[:, None, :], (NBINS, PHASES * L // F, F)).reshape(NBINS * PHASES * L)
    hi_sc = jnp.broadcast_to(
        lt_last[None, :], (PHASES * L // F, F)).reshape(PHASES * L)
    edges_tc = jnp.broadcast_to(
        ge_t[:, None, :], (NBINS, PHASES * COLS // F, F)
    ).reshape(NBINS, PHASES, COLS)
    hi_tc = jnp.broadcast_to(
        lt_last[None, :], (PHASES * COLS // F, F)).reshape(PHASES, COLS)

    x_sc = lax.slice(x2, (TC_ROWS, 0), (ROWS, COLS)).reshape(SC_TOTAL)

    mesh = plsc.VectorSubcoreMesh(core_axis_name="c", subcore_axis_name="s")
    sc_run = pl.kernel(
        _sc_kernel,
        mesh=mesh,
        out_type=jax.ShapeDtypeStruct((SC_TOTAL,), jnp.int32),
        scratch_types=[
            pltpu.VMEM((2, PIECE), jnp.float32),
            pltpu.VMEM((2, PIECE), jnp.int32),
            pltpu.VMEM((NBINS * PHASES * L,), jnp.float32),
            pltpu.VMEM((PHASES * L,), jnp.float32),
            pltpu.SemaphoreType.DMA,
            pltpu.SemaphoreType.DMA,
        ],
    )
    sc_out = sc_run(x_sc, edges_sc, hi_sc)

    tc_out = pl.pallas_call(
        _tc_kernel,
        grid=(TC_GRID,),
        in_specs=[
            pl.BlockSpec((TC_BLOCK_ROWS, COLS), lambda i: (i, 0)),
            pl.BlockSpec((NBINS, PHASES, COLS), lambda i: (0, 0, 0)),
            pl.BlockSpec((PHASES, COLS), lambda i: (0, 0)),
        ],
        out_specs=pl.BlockSpec((TC_BLOCK_ROWS, COLS), lambda i: (i, 0)),
        out_shape=jax.ShapeDtypeStruct((ROWS, COLS), jnp.int8),
    )(x2, edges_tc, hi_tc)

    out2 = lax.dynamic_update_slice(
        tc_out, sc_out.astype(jnp.int8).reshape(SC_ROWS, COLS), (TC_ROWS, 0))
    return out2.reshape(N, F).astype(jnp.int32)


# speedup vs baseline: 1.0167x; 1.0167x over previous
import jax
import jax.numpy as jnp
from jax import lax
from jax.experimental import pallas as pl
from jax.experimental.pallas import tpu as pltpu, tpu_sc as plsc

N = 262144
F = 26
NBINS = 16
L = 16
PHASES = 13
TOTAL = N * F

COLS = 1024
ROWS = TOTAL // COLS
TC_BLOCK_ROWS = 416
SC_ROWS = 416
TC_ROWS = ROWS - SC_ROWS
TC_GRID = TC_ROWS // TC_BLOCK_ROWS

NWORK = 32
SC_TOTAL = SC_ROWS * COLS
SC_PER_W = SC_TOTAL // NWORK
PIECE = 3328
NP = SC_PER_W // PIECE


def _sc_kernel(x_hbm, edges_hbm, hi_hbm, out_hbm, xb, ob, ev, hv, insem, outsem):
    nc = lax.axis_size("c")
    wid = lax.axis_index("s") * nc + lax.axis_index("c")
    pltpu.sync_copy(edges_hbm, ev)
    pltpu.sync_copy(hi_hbm, hv)
    wbase = wid * SC_PER_W

    def compute_piece(buf, xbuf, obuf):
        def group_body(g, carry):
            goff = g * (PHASES * L)
            for p in range(PHASES):
                off = goff + p * L
                xv = xbuf[buf, pl.ds(off, L)]
                cnt = jnp.zeros((L,), jnp.int32)
                for b in range(1, NBINS):
                    cnt = cnt + jnp.where(xv >= ev[pl.ds((b * PHASES + p) * L, L)], 1, 0)
                idx = jnp.where(xv < hv[pl.ds(p * L, L)], cnt, 0)
                obuf[buf, pl.ds(off, L)] = idx
            return carry

        lax.fori_loop(0, PIECE // (PHASES * L), group_body, 0)

    in_h = [None] * NP
    out_h = [None] * NP
    in_h[0] = pltpu.async_copy(x_hbm.at[pl.ds(wbase, PIECE)], xb.at[0], insem)
    for r in range(NP):
        if r + 1 < NP:
            in_h[r + 1] = pltpu.async_copy(
                x_hbm.at[pl.ds(wbase + (r + 1) * PIECE, PIECE)],
                xb.at[(r + 1) % 2], insem)
        in_h[r].wait()
        if r >= 2:
            out_h[r - 2].wait()
        compute_piece(r % 2, xb, ob)
        out_h[r] = pltpu.async_copy(
            ob.at[r % 2], out_hbm.at[pl.ds(wbase + r * PIECE, PIECE)], outsem)
    for r in range(max(NP - 2, 0), NP):
        out_h[r].wait()


def _tc_kernel(x_ref, e_ref, hi_ref, o_ref):
    x = x_ref[...].reshape(TC_BLOCK_ROWS // PHASES, PHASES, COLS)
    cnt = jnp.zeros(x.shape, jnp.int32)
    for b in range(1, NBINS):
        cnt = cnt + jnp.where(x >= e_ref[b][None], 1, 0)
    idx = jnp.where(x < hi_ref[...][None], cnt, 0)
    o_ref[...] = idx.reshape(TC_BLOCK_ROWS, COLS).astype(jnp.int8)


def kernel(x, ge_tensor, lt_tensor):
    x2 = x.astype(jnp.float32).reshape(ROWS, COLS)

    ge_t = ge_tensor.T
    lt_last = lt_tensor[:, NBINS - 1]

    edges_sc = jnp.broadcast_to(
        ge_t[:, None, :], (NBINS, PHASES * L // F, F)).reshape(NBINS * PHASES * L)
    hi_sc = jnp.broadcast_to(
        lt_last[None, :], (PHASES * L // F, F)).reshape(PHASES * L)
    edges_tc = jnp.broadcast_to(
        ge_t[:, None, :], (NBINS, PHASES * COLS // F, F)
    ).reshape(NBINS, PHASES, COLS)
    hi_tc = jnp.broadcast_to(
        lt_last[None, :], (PHASES * COLS // F, F)).reshape(PHASES, COLS)

    x_sc = lax.slice(x2, (TC_ROWS, 0), (ROWS, COLS)).reshape(SC_TOTAL)

    mesh = plsc.VectorSubcoreMesh(core_axis_name="c", subcore_axis_name="s")
    sc_run = pl.kernel(
        _sc_kernel,
        mesh=mesh,
        out_type=jax.ShapeDtypeStruct((SC_TOTAL,), jnp.int32),
        scratch_types=[
            pltpu.VMEM((2, PIECE), jnp.float32),
            pltpu.VMEM((2, PIECE), jnp.int32),
            pltpu.VMEM((NBINS * PHASES * L,), jnp.float32),
            pltpu.VMEM((PHASES * L,), jnp.float32),
            pltpu.SemaphoreType.DMA,
            pltpu.SemaphoreType.DMA,
        ],
    )
    sc_out = sc_run(x_sc, edges_sc, hi_sc)

    tc_out = pl.pallas_call(
        _tc_kernel,
        grid=(TC_GRID,),
        in_specs=[
            pl.BlockSpec((TC_BLOCK_ROWS, COLS), lambda i: (i, 0)),
            pl.BlockSpec((NBINS, PHASES, COLS), lambda i: (0, 0, 0)),
            pl.BlockSpec((PHASES, COLS), lambda i: (0, 0)),
        ],
        out_specs=pl.BlockSpec((TC_BLOCK_ROWS, COLS), lambda i: (i, 0)),
        out_shape=jax.ShapeDtypeStruct((ROWS, COLS), jnp.int8),
    )(x2, edges_tc, hi_tc)

    out2 = lax.dynamic_update_slice(
        tc_out, sc_out.astype(jnp.int8).reshape(SC_ROWS, COLS), (TC_ROWS, 0))
    out3 = lax.optimization_barrier(out2.reshape(N, F))
    return out3.astype(jnp.int32)


# speedup vs baseline: 1.0213x; 1.0046x over previous
import jax
import jax.numpy as jnp
from jax import lax
from jax.experimental import pallas as pl
from jax.experimental.pallas import tpu as pltpu, tpu_sc as plsc

N = 262144
F = 26
NBINS = 16
L = 16
PHASES = 13
TOTAL = N * F

COLS = 1024
ROWS = TOTAL // COLS
TC_BLOCK_ROWS = 416
SC_ROWS = 832
TC_ROWS = ROWS - SC_ROWS
TC_GRID = TC_ROWS // TC_BLOCK_ROWS

NWORK = 32
SC_TOTAL = SC_ROWS * COLS
SC_PER_W = SC_TOTAL // NWORK
PIECE = 3328
NP = SC_PER_W // PIECE


def _sc_kernel(x_hbm, edges_hbm, hi_hbm, out_hbm, xb, ob, ev, hv, insem, outsem):
    nc = lax.axis_size("c")
    wid = lax.axis_index("s") * nc + lax.axis_index("c")
    pltpu.sync_copy(edges_hbm, ev)
    pltpu.sync_copy(hi_hbm, hv)
    wbase = wid * SC_PER_W

    def compute_piece(buf, xbuf, obuf):
        def group_body(g, carry):
            goff = g * (PHASES * L)
            for p in range(PHASES):
                off = goff + p * L
                xv = xbuf[buf, pl.ds(off, L)]
                cnt = jnp.zeros((L,), jnp.int32)
                for b in range(1, NBINS):
                    cnt = cnt + jnp.where(xv >= ev[pl.ds((b * PHASES + p) * L, L)], 1, 0)
                idx = jnp.where(xv < hv[pl.ds(p * L, L)], cnt, 0)
                obuf[buf, pl.ds(off, L)] = idx
            return carry

        lax.fori_loop(0, PIECE // (PHASES * L), group_body, 0)

    in_h = [None] * NP
    out_h = [None] * NP
    in_h[0] = pltpu.async_copy(x_hbm.at[pl.ds(wbase, PIECE)], xb.at[0], insem)
    for r in range(NP):
        if r + 1 < NP:
            in_h[r + 1] = pltpu.async_copy(
                x_hbm.at[pl.ds(wbase + (r + 1) * PIECE, PIECE)],
                xb.at[(r + 1) % 2], insem)
        in_h[r].wait()
        if r >= 2:
            out_h[r - 2].wait()
        compute_piece(r % 2, xb, ob)
        out_h[r] = pltpu.async_copy(
            ob.at[r % 2], out_hbm.at[pl.ds(wbase + r * PIECE, PIECE)], outsem)
    for r in range(max(NP - 2, 0), NP):
        out_h[r].wait()


def _tc_kernel(x_ref, e_ref, hi_ref, o_ref):
    x = x_ref[...].reshape(TC_BLOCK_ROWS // PHASES, PHASES, COLS)
    cnt = jnp.zeros(x.shape, jnp.int32)
    for b in range(1, NBINS):
        cnt = cnt + jnp.where(x >= e_ref[b][None], 1, 0)
    idx = jnp.where(x < hi_ref[...][None], cnt, 0)
    o_ref[...] = idx.reshape(TC_BLOCK_ROWS, COLS)


def kernel(x, ge_tensor, lt_tensor):
    x2 = x.astype(jnp.float32).reshape(ROWS, COLS)

    ge_t = ge_tensor.T
    lt_last = lt_tensor[:, NBINS - 1]

    edges_sc = jnp.broadcast_to(
        ge_t[:, None, :], (NBINS, PHASES * L // F, F)).reshape(NBINS * PHASES * L)
    hi_sc = jnp.broadcast_to(
        lt_last[None, :], (PHASES * L // F, F)).reshape(PHASES * L)
    edges_tc = jnp.broadcast_to(
        ge_t[:, None, :], (NBINS, PHASES * COLS // F, F)
    ).reshape(NBINS, PHASES, COLS)
    hi_tc = jnp.broadcast_to(
        lt_last[None, :], (PHASES * COLS // F, F)).reshape(PHASES, COLS)

    x_sc = lax.slice(x2, (TC_ROWS, 0), (ROWS, COLS)).reshape(SC_TOTAL)

    mesh = plsc.VectorSubcoreMesh(core_axis_name="c", subcore_axis_name="s")
    sc_run = pl.kernel(
        _sc_kernel,
        mesh=mesh,
        out_type=jax.ShapeDtypeStruct((SC_TOTAL,), jnp.int32),
        scratch_types=[
            pltpu.VMEM((2, PIECE), jnp.float32),
            pltpu.VMEM((2, PIECE), jnp.int32),
            pltpu.VMEM((NBINS * PHASES * L,), jnp.float32),
            pltpu.VMEM((PHASES * L,), jnp.float32),
            pltpu.SemaphoreType.DMA,
            pltpu.SemaphoreType.DMA,
        ],
    )
    sc_out = sc_run(x_sc, edges_sc, hi_sc)

    tc_out = pl.pallas_call(
        _tc_kernel,
        grid=(TC_GRID,),
        in_specs=[
            pl.BlockSpec((TC_BLOCK_ROWS, COLS), lambda i: (i, 0)),
            pl.BlockSpec((NBINS, PHASES, COLS), lambda i: (0, 0, 0)),
            pl.BlockSpec((PHASES, COLS), lambda i: (0, 0)),
        ],
        out_specs=pl.BlockSpec((TC_BLOCK_ROWS, COLS), lambda i: (i, 0)),
        out_shape=jax.ShapeDtypeStruct((ROWS, COLS), jnp.int32),
    )(x2, edges_tc, hi_tc)

    out2 = lax.dynamic_update_slice(
        tc_out, sc_out.reshape(SC_ROWS, COLS), (TC_ROWS, 0))
    return out2.reshape(N, F)
